# baseline (speedup 1.0000x reference)
import functools

import jax
import jax.numpy as jnp
from jax.experimental import pallas as pl
from jax.experimental.pallas import tpu as pltpu
from jax.experimental.pallas import tpu_sc as plsc

_EPS = 1e-5
_WIN = 128
_NWORK = 32
_BLK = 200
_BLK3 = 1000
_NCHUNK = 5


def _sc_gather(table, idx_pad, epad):
    a = table.shape[1]
    mesh = plsc.VectorSubcoreMesh(core_axis_name="core", subcore_axis_name="subcore")

    steps = epad // _WIN // _NWORK
    assert steps * _WIN * _NWORK == epad

    @functools.partial(
        pl.kernel,
        out_type=jax.ShapeDtypeStruct((epad, a), table.dtype),
        mesh=mesh,
    )
    def gather_kernel(x_hbm, i_hbm, o_hbm):
        def body(i_vmem, o_vmem):
            pltpu.sync_copy(x_hbm.at[i_vmem.at[0]], o_vmem)

        pltpu.emit_pipeline(
            body,
            grid=(_NWORK, steps),
            in_specs=[pl.BlockSpec((1, _WIN), lambda i, j: (0, i * steps + j))],
            out_specs=[pl.BlockSpec((_WIN, a), lambda i, j: (i * steps + j, 0))],
            core_axis_name=("core", "subcore"),
            dimension_semantics=(pltpu.PARALLEL, pltpu.ARBITRARY),
        )(i_hbm, o_hbm)

    return gather_kernel(table, idx_pad)


def _p1_kernel(x, w, b2d):
    n, a = x.shape
    c = w.shape[1]
    blk = _BLK3

    def body(x_ref, w_ref, b_ref, o_ref):
        w1 = w_ref[0:a, :]
        o_ref[...] = (
            jnp.dot(x_ref[...], w1, preferred_element_type=jnp.float32)
            + b_ref[...]
        )

    return pl.pallas_call(
        body,
        grid=(n // blk,),
        in_specs=[
            pl.BlockSpec((blk, a), lambda i: (i, 0)),
            pl.BlockSpec(w.shape, lambda i: (0, 0)),
            pl.BlockSpec(b2d.shape, lambda i: (0, 0)),
        ],
        out_specs=pl.BlockSpec((blk, c), lambda i: (i, 0)),
        out_shape=jax.ShapeDtypeStruct((n, c), jnp.float32),
    )(x, w, b2d)


def _edge_rows(g_ref, nb_ref, p1_ref, w_ref, a, m, blk):
    w2 = w_ref[a : 2 * a, :].astype(jnp.bfloat16)
    w3 = w_ref[2 * a :, :].astype(jnp.bfloat16)
    g = jnp.dot(
        g_ref[...].astype(jnp.bfloat16), w2, preferred_element_type=jnp.float32
    )
    g = g + jnp.dot(
        nb_ref[...].astype(jnp.bfloat16), w3, preferred_element_type=jnp.float32
    )
    g3 = g.reshape(blk, m, 2 * a) + p1_ref[...][:, None, :]
    return g3


def _stats_pass(g_rows, nbr2, p1, w, nc, m, a, boff):
    blk = _BLK
    eblk = blk * m
    grid = nc // blk

    def body(g_ref, nb_ref, p1_ref, w_ref, o_ref):
        g3 = _edge_rows(g_ref, nb_ref, p1_ref, w_ref, a, m, blk)

        @pl.when(pl.program_id(0) == 0)
        def _():
            o_ref[...] = jnp.zeros_like(o_ref)

        o_ref[0:1, :] += jnp.sum(g3, axis=(0, 1))[None, :]
        o_ref[1:2, :] += jnp.sum(g3 * g3, axis=(0, 1))[None, :]

    return pl.pallas_call(
        body,
        grid=(grid,),
        in_specs=[
            pl.BlockSpec((eblk, a), lambda i: (i, 0)),
            pl.BlockSpec((eblk, nbr2.shape[1]), lambda i, o=boff: (o + i, 0)),
            pl.BlockSpec((blk, 2 * a), lambda i, o=boff: (o + i, 0)),
            pl.BlockSpec(w.shape, lambda i: (0, 0)),
        ],
        out_specs=pl.BlockSpec((8, 2 * a), lambda i: (0, 0)),
        out_shape=jax.ShapeDtypeStruct((8, 2 * a), jnp.float32),
    )(g_rows, nbr2, p1, w)


def _gate_pass(g_rows, nbr2, p1, w, st1, gamma1, beta1, nc, m, a, boff, cnt):
    blk = _BLK
    eblk = blk * m
    grid = nc // blk

    def body(g_ref, nb_ref, p1_ref, w_ref, st_ref, ga_ref, be_ref, s_ref, o2_ref):
        st = jnp.sum(st_ref[...].reshape(_NCHUNK, 8, 2 * a), axis=0)
        mean = st[0:1, :] / cnt
        var = st[1:2, :] / cnt - mean * mean
        scale = ga_ref[...] * jax.lax.rsqrt(var + _EPS)
        shift = be_ref[...] - mean * scale

        g3 = _edge_rows(g_ref, nb_ref, p1_ref, w_ref, a, m, blk)
        y = g3 * scale[None, :, :] + shift[None, :, :]
        filt = jax.nn.sigmoid(y[:, :, :a])
        core = jax.nn.softplus(y[:, :, a:])
        s_blk = jnp.sum(filt * core, axis=1)
        s_ref[...] = s_blk

        @pl.when(pl.program_id(0) == 0)
        def _():
            o2_ref[...] = jnp.zeros_like(o2_ref)

        o2_ref[0:1, :] += jnp.sum(s_blk, axis=0)[None, :]
        o2_ref[1:2, :] += jnp.sum(s_blk * s_blk, axis=0)[None, :]

    return pl.pallas_call(
        body,
        grid=(grid,),
        in_specs=[
            pl.BlockSpec((eblk, a), lambda i: (i, 0)),
            pl.BlockSpec((eblk, nbr2.shape[1]), lambda i, o=boff: (o + i, 0)),
            pl.BlockSpec((blk, 2 * a), lambda i, o=boff: (o + i, 0)),
            pl.BlockSpec(w.shape, lambda i: (0, 0)),
            pl.BlockSpec(st1.shape, lambda i: (0, 0)),
            pl.BlockSpec((1, 2 * a), lambda i: (0, 0)),
            pl.BlockSpec((1, 2 * a), lambda i: (0, 0)),
        ],
        out_specs=[
            pl.BlockSpec((blk, a), lambda i: (i, 0)),
            pl.BlockSpec((8, a), lambda i: (0, 0)),
        ],
        out_shape=[
            jax.ShapeDtypeStruct((nc, a), jnp.float32),
            jax.ShapeDtypeStruct((8, a), jnp.float32),
        ],
    )(g_rows, nbr2, p1, w, st1, gamma1, beta1)


def _final_pass(x, s, st2, gamma2, beta2):
    n, a = x.shape
    blk = _BLK3
    cnt = float(n)

    def body(x_ref, s_ref, st_ref, ga_ref, be_ref, o_ref):
        st = jnp.sum(st_ref[...].reshape(_NCHUNK, 8, a), axis=0)
        mean = st[0:1, :] / cnt
        var = st[1:2, :] / cnt - mean * mean
        scale = ga_ref[...] * jax.lax.rsqrt(var + _EPS)
        shift = be_ref[...] - mean * scale
        o_ref[...] = jax.nn.softplus(x_ref[...] + s_ref[...] * scale + shift)

    return pl.pallas_call(
        body,
        grid=(n // blk,),
        in_specs=[
            pl.BlockSpec((blk, a), lambda i: (i, 0)),
            pl.BlockSpec((blk, a), lambda i: (i, 0)),
            pl.BlockSpec(st2.shape, lambda i: (0, 0)),
            pl.BlockSpec((1, a), lambda i: (0, 0)),
            pl.BlockSpec((1, a), lambda i: (0, 0)),
        ],
        out_specs=pl.BlockSpec((blk, a), lambda i: (i, 0)),
        out_shape=jax.ShapeDtypeStruct((n, a), jnp.float32),
    )(x, s, st2, gamma2, beta2)


def kernel(atom_in_fea, nbr_fea, nbr_fea_idx, W, b, gamma1, beta1, gamma2, beta2):
    n, m = nbr_fea_idx.shape
    a = atom_in_fea.shape[1]
    e = n * m
    nc = n // _NCHUNK
    ec = nc * m
    boff_step = nc // _BLK

    grain = _WIN * _NWORK
    epad = ((ec + grain - 1) // grain) * grain
    flat_idx = nbr_fea_idx.reshape(1, e)
    pad = jnp.zeros((1, epad - ec), jnp.int32) if epad != ec else None

    nbr2 = nbr_fea.reshape(e, nbr_fea.shape[2])
    b2d = b.reshape(1, 2 * a)
    g1 = gamma1.reshape(1, 2 * a)
    b1 = beta1.reshape(1, 2 * a)
    g2 = gamma2.reshape(1, a)
    b2 = beta2.reshape(1, a)

    p1 = _p1_kernel(atom_in_fea, W, b2d)

    g_chunks = []
    for c in range(_NCHUNK):
        idx_c = flat_idx[:, c * ec : (c + 1) * ec]
        if pad is not None:
            idx_c = jnp.concatenate([idx_c, pad], axis=1)
        g_chunks.append(_sc_gather(atom_in_fea, idx_c, epad))

    st1_parts = [
        _stats_pass(g_chunks[c], nbr2, p1, W, nc, m, a, c * boff_step)
        for c in range(_NCHUNK)
    ]
    st1 = jnp.concatenate(st1_parts, axis=0)

    s_parts = []
    st2_parts = []
    for c in range(_NCHUNK):
        s_c, st2_c = _gate_pass(
            g_chunks[c], nbr2, p1, W, st1, g1, b1, nc, m, a, c * boff_step,
            float(e),
        )
        s_parts.append(s_c)
        st2_parts.append(st2_c)
    s = jnp.concatenate(s_parts, axis=0)
    st2 = jnp.concatenate(st2_parts, axis=0)

    return _final_pass(atom_in_fea, s, st2, g2, b2)


# speedup vs baseline: 2.0527x; 2.0527x over previous
import functools

import jax
import jax.numpy as jnp
from jax.experimental import pallas as pl
from jax.experimental.pallas import tpu as pltpu
from jax.experimental.pallas import tpu_sc as plsc

_EPS = 1e-5
_WIN = 128
_NWORK = 32
_BLK = 200
_BLK3 = 1000
_NCHUNK = 1


def _sc_gather(table, idx_pad, epad):
    a = table.shape[1]
    mesh = plsc.VectorSubcoreMesh(core_axis_name="core", subcore_axis_name="subcore")

    steps = epad // _WIN // _NWORK
    assert steps * _WIN * _NWORK == epad

    @functools.partial(
        pl.kernel,
        out_type=jax.ShapeDtypeStruct((epad, a), table.dtype),
        mesh=mesh,
        scratch_types=[pltpu.VMEM_SHARED(table.shape, table.dtype)],
    )
    def gather_kernel(x_hbm, i_hbm, o_hbm, tab_shared):
        @pl.when(jax.lax.axis_index("subcore") == 0)
        def _():
            pltpu.sync_copy(x_hbm, tab_shared)

        plsc.subcore_barrier()

        def body(i_vmem, o_vmem):
            pltpu.sync_copy(tab_shared.at[i_vmem.at[0]], o_vmem)

        pltpu.emit_pipeline(
            body,
            grid=(_NWORK, steps),
            in_specs=[pl.BlockSpec((1, _WIN), lambda i, j: (0, i * steps + j))],
            out_specs=[pl.BlockSpec((_WIN, a), lambda i, j: (i * steps + j, 0))],
            core_axis_name=("core", "subcore"),
            dimension_semantics=(pltpu.PARALLEL, pltpu.ARBITRARY),
        )(i_hbm, o_hbm)

    return gather_kernel(table, idx_pad)


def _p1_kernel(x, w, b2d):
    n, a = x.shape
    c = w.shape[1]
    blk = _BLK3

    def body(x_ref, w_ref, b_ref, o_ref):
        w1 = w_ref[0:a, :]
        o_ref[...] = (
            jnp.dot(x_ref[...], w1, preferred_element_type=jnp.float32)
            + b_ref[...]
        )

    return pl.pallas_call(
        body,
        grid=(n // blk,),
        in_specs=[
            pl.BlockSpec((blk, a), lambda i: (i, 0)),
            pl.BlockSpec(w.shape, lambda i: (0, 0)),
            pl.BlockSpec(b2d.shape, lambda i: (0, 0)),
        ],
        out_specs=pl.BlockSpec((blk, c), lambda i: (i, 0)),
        out_shape=jax.ShapeDtypeStruct((n, c), jnp.float32),
    )(x, w, b2d)


def _edge_rows(g_ref, nb_ref, p1_ref, w_ref, a, m, blk):
    w2 = w_ref[a : 2 * a, :].astype(jnp.bfloat16)
    w3 = w_ref[2 * a :, :].astype(jnp.bfloat16)
    g = jnp.dot(
        g_ref[...].astype(jnp.bfloat16), w2, preferred_element_type=jnp.float32
    )
    g = g + jnp.dot(
        nb_ref[...].astype(jnp.bfloat16), w3, preferred_element_type=jnp.float32
    )
    g3 = g.reshape(blk, m, 2 * a) + p1_ref[...][:, None, :]
    return g3


def _stats_pass(g_rows, nbr2, p1, w, nc, m, a, boff):
    blk = _BLK
    eblk = blk * m
    grid = nc // blk

    def body(g_ref, nb_ref, p1_ref, w_ref, o_ref):
        g3 = _edge_rows(g_ref, nb_ref, p1_ref, w_ref, a, m, blk)

        @pl.when(pl.program_id(0) == 0)
        def _():
            o_ref[...] = jnp.zeros_like(o_ref)

        o_ref[0:1, :] += jnp.sum(g3, axis=(0, 1))[None, :]
        o_ref[1:2, :] += jnp.sum(g3 * g3, axis=(0, 1))[None, :]

    return pl.pallas_call(
        body,
        grid=(grid,),
        in_specs=[
            pl.BlockSpec((eblk, a), lambda i: (i, 0)),
            pl.BlockSpec((eblk, nbr2.shape[1]), lambda i, o=boff: (o + i, 0)),
            pl.BlockSpec((blk, 2 * a), lambda i, o=boff: (o + i, 0)),
            pl.BlockSpec(w.shape, lambda i: (0, 0)),
        ],
        out_specs=pl.BlockSpec((8, 2 * a), lambda i: (0, 0)),
        out_shape=jax.ShapeDtypeStruct((8, 2 * a), jnp.float32),
    )(g_rows, nbr2, p1, w)


def _gate_pass(g_rows, nbr2, p1, w, st1, gamma1, beta1, nc, m, a, boff, cnt):
    blk = _BLK
    eblk = blk * m
    grid = nc // blk

    def body(g_ref, nb_ref, p1_ref, w_ref, st_ref, ga_ref, be_ref, s_ref, o2_ref):
        st = jnp.sum(st_ref[...].reshape(_NCHUNK, 8, 2 * a), axis=0)
        mean = st[0:1, :] / cnt
        var = st[1:2, :] / cnt - mean * mean
        scale = ga_ref[...] * jax.lax.rsqrt(var + _EPS)
        shift = be_ref[...] - mean * scale

        g3 = _edge_rows(g_ref, nb_ref, p1_ref, w_ref, a, m, blk)
        y = g3 * scale[None, :, :] + shift[None, :, :]
        filt = jax.nn.sigmoid(y[:, :, :a])
        core = jax.nn.softplus(y[:, :, a:])
        s_blk = jnp.sum(filt * core, axis=1)
        s_ref[...] = s_blk

        @pl.when(pl.program_id(0) == 0)
        def _():
            o2_ref[...] = jnp.zeros_like(o2_ref)

        o2_ref[0:1, :] += jnp.sum(s_blk, axis=0)[None, :]
        o2_ref[1:2, :] += jnp.sum(s_blk * s_blk, axis=0)[None, :]

    return pl.pallas_call(
        body,
        grid=(grid,),
        in_specs=[
            pl.BlockSpec((eblk, a), lambda i: (i, 0)),
            pl.BlockSpec((eblk, nbr2.shape[1]), lambda i, o=boff: (o + i, 0)),
            pl.BlockSpec((blk, 2 * a), lambda i, o=boff: (o + i, 0)),
            pl.BlockSpec(w.shape, lambda i: (0, 0)),
            pl.BlockSpec(st1.shape, lambda i: (0, 0)),
            pl.BlockSpec((1, 2 * a), lambda i: (0, 0)),
            pl.BlockSpec((1, 2 * a), lambda i: (0, 0)),
        ],
        out_specs=[
            pl.BlockSpec((blk, a), lambda i: (i, 0)),
            pl.BlockSpec((8, a), lambda i: (0, 0)),
        ],
        out_shape=[
            jax.ShapeDtypeStruct((nc, a), jnp.float32),
            jax.ShapeDtypeStruct((8, a), jnp.float32),
        ],
    )(g_rows, nbr2, p1, w, st1, gamma1, beta1)


def _final_pass(x, s, st2, gamma2, beta2):
    n, a = x.shape
    blk = _BLK3
    cnt = float(n)

    def body(x_ref, s_ref, st_ref, ga_ref, be_ref, o_ref):
        st = jnp.sum(st_ref[...].reshape(_NCHUNK, 8, a), axis=0)
        mean = st[0:1, :] / cnt
        var = st[1:2, :] / cnt - mean * mean
        scale = ga_ref[...] * jax.lax.rsqrt(var + _EPS)
        shift = be_ref[...] - mean * scale
        o_ref[...] = jax.nn.softplus(x_ref[...] + s_ref[...] * scale + shift)

    return pl.pallas_call(
        body,
        grid=(n // blk,),
        in_specs=[
            pl.BlockSpec((blk, a), lambda i: (i, 0)),
            pl.BlockSpec((blk, a), lambda i: (i, 0)),
            pl.BlockSpec(st2.shape, lambda i: (0, 0)),
            pl.BlockSpec((1, a), lambda i: (0, 0)),
            pl.BlockSpec((1, a), lambda i: (0, 0)),
        ],
        out_specs=pl.BlockSpec((blk, a), lambda i: (i, 0)),
        out_shape=jax.ShapeDtypeStruct((n, a), jnp.float32),
    )(x, s, st2, gamma2, beta2)


def kernel(atom_in_fea, nbr_fea, nbr_fea_idx, W, b, gamma1, beta1, gamma2, beta2):
    n, m = nbr_fea_idx.shape
    a = atom_in_fea.shape[1]
    e = n * m
    nc = n // _NCHUNK
    ec = nc * m
    boff_step = nc // _BLK

    grain = _WIN * _NWORK
    epad = ((ec + grain - 1) // grain) * grain
    flat_idx = nbr_fea_idx.reshape(1, e)
    pad = jnp.zeros((1, epad - ec), jnp.int32) if epad != ec else None

    nbr2 = nbr_fea.reshape(e, nbr_fea.shape[2])
    b2d = b.reshape(1, 2 * a)
    g1 = gamma1.reshape(1, 2 * a)
    b1 = beta1.reshape(1, 2 * a)
    g2 = gamma2.reshape(1, a)
    b2 = beta2.reshape(1, a)

    p1 = _p1_kernel(atom_in_fea, W, b2d)

    g_chunks = []
    for c in range(_NCHUNK):
        idx_c = flat_idx[:, c * ec : (c + 1) * ec]
        if pad is not None:
            idx_c = jnp.concatenate([idx_c, pad], axis=1)
        g_chunks.append(_sc_gather(atom_in_fea, idx_c, epad))

    st1_parts = [
        _stats_pass(g_chunks[c], nbr2, p1, W, nc, m, a, c * boff_step)
        for c in range(_NCHUNK)
    ]
    st1 = jnp.concatenate(st1_parts, axis=0)

    s_parts = []
    st2_parts = []
    for c in range(_NCHUNK):
        s_c, st2_c = _gate_pass(
            g_chunks[c], nbr2, p1, W, st1, g1, b1, nc, m, a, c * boff_step,
            float(e),
        )
        s_parts.append(s_c)
        st2_parts.append(st2_c)
    s = jnp.concatenate(s_parts, axis=0)
    st2 = jnp.concatenate(st2_parts, axis=0)

    return _final_pass(atom_in_fea, s, st2, g2, b2)


# speedup vs baseline: 2.1816x; 1.0628x over previous
import functools

import jax
import jax.numpy as jnp
from jax.experimental import pallas as pl
from jax.experimental.pallas import tpu as pltpu
from jax.experimental.pallas import tpu_sc as plsc

_EPS = 1e-5
_WIN = 128
_NWORK = 32
_BLK = 200
_BLK3 = 1000
_NCHUNK = 1


def _sc_gather(table, idx_pad, epad):
    a = table.shape[1]
    mesh = plsc.VectorSubcoreMesh(core_axis_name="core", subcore_axis_name="subcore")

    steps = epad // _WIN // _NWORK
    assert steps * _WIN * _NWORK == epad

    @functools.partial(
        pl.kernel,
        out_type=jax.ShapeDtypeStruct((epad, a), table.dtype),
        mesh=mesh,
        scratch_types=[pltpu.VMEM_SHARED(table.shape, table.dtype)],
    )
    def gather_kernel(x_hbm, i_hbm, o_hbm, tab_shared):
        @pl.when(jax.lax.axis_index("subcore") == 0)
        def _():
            pltpu.sync_copy(x_hbm, tab_shared)

        plsc.subcore_barrier()

        def body(i_vmem, o_vmem):
            pltpu.sync_copy(tab_shared.at[i_vmem.at[0]], o_vmem)

        pltpu.emit_pipeline(
            body,
            grid=(_NWORK, steps),
            in_specs=[pl.BlockSpec((1, _WIN), lambda i, j: (0, i * steps + j))],
            out_specs=[pl.BlockSpec((_WIN, a), lambda i, j: (i * steps + j, 0))],
            core_axis_name=("core", "subcore"),
            dimension_semantics=(pltpu.PARALLEL, pltpu.ARBITRARY),
        )(i_hbm, o_hbm)

    return gather_kernel(table, idx_pad)


def _p1_kernel(x, w, b2d):
    n, a = x.shape
    c = w.shape[1]
    blk = _BLK3

    def body(x_ref, w_ref, b_ref, o_ref):
        w1 = w_ref[0:a, :]
        o_ref[...] = (
            jnp.dot(x_ref[...], w1, preferred_element_type=jnp.float32)
            + b_ref[...]
        )

    return pl.pallas_call(
        body,
        grid=(n // blk,),
        in_specs=[
            pl.BlockSpec((blk, a), lambda i: (i, 0)),
            pl.BlockSpec(w.shape, lambda i: (0, 0)),
            pl.BlockSpec(b2d.shape, lambda i: (0, 0)),
        ],
        out_specs=pl.BlockSpec((blk, c), lambda i: (i, 0)),
        out_shape=jax.ShapeDtypeStruct((n, c), jnp.float32),
    )(x, w, b2d)


_LOG2E = 1.4426950408889634
_LN2 = 0.6931471805599453


def _sigmoid(x):
    return 0.5 * jnp.tanh(0.5 * x) + 0.5


def _softplus(x):
    e = jnp.exp2(-jnp.abs(x) * _LOG2E)
    return jnp.maximum(x, 0.0) + _LN2 * jnp.log2(1.0 + e)


def _edge_rows(g_ref, nb_ref, p1_ref, w_ref, a, m, blk):
    w2 = w_ref[a : 2 * a, :].astype(jnp.bfloat16)
    w3 = w_ref[2 * a :, :].astype(jnp.bfloat16)
    g = jnp.dot(
        g_ref[...].astype(jnp.bfloat16), w2, preferred_element_type=jnp.float32
    )
    g = g + jnp.dot(
        nb_ref[...].astype(jnp.bfloat16), w3, preferred_element_type=jnp.float32
    )
    g3 = g.reshape(blk, m, 2 * a) + p1_ref[...][:, None, :]
    return g3


def _stats_pass(g_rows, nbr2, p1, w, nc, m, a, boff):
    blk = _BLK
    eblk = blk * m
    grid = nc // blk

    def body(g_ref, nb_ref, p1_ref, w_ref, o_ref):
        g3 = _edge_rows(g_ref, nb_ref, p1_ref, w_ref, a, m, blk)

        @pl.when(pl.program_id(0) == 0)
        def _():
            o_ref[...] = jnp.zeros_like(o_ref)

        o_ref[0:1, :] += jnp.sum(g3, axis=(0, 1))[None, :]
        o_ref[1:2, :] += jnp.sum(g3 * g3, axis=(0, 1))[None, :]

    return pl.pallas_call(
        body,
        grid=(grid,),
        in_specs=[
            pl.BlockSpec((eblk, a), lambda i: (i, 0)),
            pl.BlockSpec((eblk, nbr2.shape[1]), lambda i, o=boff: (o + i, 0)),
            pl.BlockSpec((blk, 2 * a), lambda i, o=boff: (o + i, 0)),
            pl.BlockSpec(w.shape, lambda i: (0, 0)),
        ],
        out_specs=pl.BlockSpec((8, 2 * a), lambda i: (0, 0)),
        out_shape=jax.ShapeDtypeStruct((8, 2 * a), jnp.float32),
    )(g_rows, nbr2, p1, w)


def _gate_pass(g_rows, nbr2, p1, w, st1, gamma1, beta1, nc, m, a, boff, cnt):
    blk = _BLK
    eblk = blk * m
    grid = nc // blk

    def body(g_ref, nb_ref, p1_ref, w_ref, st_ref, ga_ref, be_ref, s_ref, o2_ref):
        st = jnp.sum(st_ref[...].reshape(_NCHUNK, 8, 2 * a), axis=0)
        mean = st[0:1, :] / cnt
        var = st[1:2, :] / cnt - mean * mean
        scale = ga_ref[...] * jax.lax.rsqrt(var + _EPS)
        shift = be_ref[...] - mean * scale

        g3 = _edge_rows(g_ref, nb_ref, p1_ref, w_ref, a, m, blk)
        y = g3 * scale[None, :, :] + shift[None, :, :]
        filt = _sigmoid(y[:, :, :a])
        core = _softplus(y[:, :, a:])
        s_blk = jnp.sum(filt * core, axis=1)
        s_ref[...] = s_blk

        @pl.when(pl.program_id(0) == 0)
        def _():
            o2_ref[...] = jnp.zeros_like(o2_ref)

        o2_ref[0:1, :] += jnp.sum(s_blk, axis=0)[None, :]
        o2_ref[1:2, :] += jnp.sum(s_blk * s_blk, axis=0)[None, :]

    return pl.pallas_call(
        body,
        grid=(grid,),
        in_specs=[
            pl.BlockSpec((eblk, a), lambda i: (i, 0)),
            pl.BlockSpec((eblk, nbr2.shape[1]), lambda i, o=boff: (o + i, 0)),
            pl.BlockSpec((blk, 2 * a), lambda i, o=boff: (o + i, 0)),
            pl.BlockSpec(w.shape, lambda i: (0, 0)),
            pl.BlockSpec(st1.shape, lambda i: (0, 0)),
            pl.BlockSpec((1, 2 * a), lambda i: (0, 0)),
            pl.BlockSpec((1, 2 * a), lambda i: (0, 0)),
        ],
        out_specs=[
            pl.BlockSpec((blk, a), lambda i: (i, 0)),
            pl.BlockSpec((8, a), lambda i: (0, 0)),
        ],
        out_shape=[
            jax.ShapeDtypeStruct((nc, a), jnp.float32),
            jax.ShapeDtypeStruct((8, a), jnp.float32),
        ],
    )(g_rows, nbr2, p1, w, st1, gamma1, beta1)


def _final_pass(x, s, st2, gamma2, beta2):
    n, a = x.shape
    blk = _BLK3
    cnt = float(n)

    def body(x_ref, s_ref, st_ref, ga_ref, be_ref, o_ref):
        st = jnp.sum(st_ref[...].reshape(_NCHUNK, 8, a), axis=0)
        mean = st[0:1, :] / cnt
        var = st[1:2, :] / cnt - mean * mean
        scale = ga_ref[...] * jax.lax.rsqrt(var + _EPS)
        shift = be_ref[...] - mean * scale
        o_ref[...] = _softplus(x_ref[...] + s_ref[...] * scale + shift)

    return pl.pallas_call(
        body,
        grid=(n // blk,),
        in_specs=[
            pl.BlockSpec((blk, a), lambda i: (i, 0)),
            pl.BlockSpec((blk, a), lambda i: (i, 0)),
            pl.BlockSpec(st2.shape, lambda i: (0, 0)),
            pl.BlockSpec((1, a), lambda i: (0, 0)),
            pl.BlockSpec((1, a), lambda i: (0, 0)),
        ],
        out_specs=pl.BlockSpec((blk, a), lambda i: (i, 0)),
        out_shape=jax.ShapeDtypeStruct((n, a), jnp.float32),
    )(x, s, st2, gamma2, beta2)


def kernel(atom_in_fea, nbr_fea, nbr_fea_idx, W, b, gamma1, beta1, gamma2, beta2):
    n, m = nbr_fea_idx.shape
    a = atom_in_fea.shape[1]
    e = n * m
    nc = n // _NCHUNK
    ec = nc * m
    boff_step = nc // _BLK

    grain = _WIN * _NWORK
    epad = ((ec + grain - 1) // grain) * grain
    flat_idx = nbr_fea_idx.reshape(1, e)
    pad = jnp.zeros((1, epad - ec), jnp.int32) if epad != ec else None

    nbr2 = nbr_fea.reshape(e, nbr_fea.shape[2])
    b2d = b.reshape(1, 2 * a)
    g1 = gamma1.reshape(1, 2 * a)
    b1 = beta1.reshape(1, 2 * a)
    g2 = gamma2.reshape(1, a)
    b2 = beta2.reshape(1, a)

    p1 = _p1_kernel(atom_in_fea, W, b2d)

    g_chunks = []
    for c in range(_NCHUNK):
        idx_c = flat_idx[:, c * ec : (c + 1) * ec]
        if pad is not None:
            idx_c = jnp.concatenate([idx_c, pad], axis=1)
        g_chunks.append(_sc_gather(atom_in_fea, idx_c, epad))

    st1_parts = [
        _stats_pass(g_chunks[c], nbr2, p1, W, nc, m, a, c * boff_step)
        for c in range(_NCHUNK)
    ]
    st1 = jnp.concatenate(st1_parts, axis=0)

    s_parts = []
    st2_parts = []
    for c in range(_NCHUNK):
        s_c, st2_c = _gate_pass(
            g_chunks[c], nbr2, p1, W, st1, g1, b1, nc, m, a, c * boff_step,
            float(e),
        )
        s_parts.append(s_c)
        st2_parts.append(st2_c)
    s = jnp.concatenate(s_parts, axis=0)
    st2 = jnp.concatenate(st2_parts, axis=0)

    return _final_pass(atom_in_fea, s, st2, g2, b2)
